# baseline (speedup 1.0000x reference)
import functools

import jax
import jax.numpy as jnp
from jax import lax
from jax.experimental import pallas as pl
from jax.experimental.pallas import tpu as pltpu
from jax.experimental.pallas import tpu_sc as plsc

_ACQ = 1
_B, _S, _D = 1024, 4, 2048
_NC, _NS = 2, 16
_NW = _NC * _NS
_RPW = _B // _NW


@functools.partial(
    pl.kernel,
    mesh=plsc.VectorSubcoreMesh(core_axis_name="c", subcore_axis_name="s"),
    out_type=jax.ShapeDtypeStruct((_B, 1, _D), jnp.float32),
    scratch_types=[pltpu.SemaphoreType.DMA],
)
def _split(in_hbm, out_hbm, sem):
    wid = lax.axis_index("s") * _NC + lax.axis_index("c")
    base = wid * _RPW
    copies = [
        pltpu.make_async_copy(
            in_hbm.at[pl.ds(base + r, 1), pl.ds(_ACQ, 1), :],
            out_hbm.at[pl.ds(base + r, 1)],
            sem,
        )
        for r in range(_RPW)
    ]
    for c in copies:
        c.start()
    for c in copies:
        c.wait()


def kernel(inputs):
    return _split(inputs).reshape(_B, _D)


# speedup vs baseline: 8.0082x; 8.0082x over previous
import functools

import jax
import jax.numpy as jnp
from jax import lax
from jax.experimental import pallas as pl
from jax.experimental.pallas import tpu as pltpu
from jax.experimental.pallas import tpu_sc as plsc

_ACQ = 1
_B, _S, _D = 1024, 4, 2048
_NC, _NS = 2, 16
_NW = _NC * _NS
_RPW = _B // _NW


@functools.partial(
    pl.kernel,
    mesh=plsc.VectorSubcoreMesh(core_axis_name="c", subcore_axis_name="s"),
    out_type=jax.ShapeDtypeStruct((_B, 1, _D), jnp.float32),
    scratch_types=[pltpu.VMEM((_RPW, 1, _D), jnp.float32)],
)
def _split(in_hbm, out_hbm, buf_v):
    wid = lax.axis_index("s") * _NC + lax.axis_index("c")
    base = wid * _RPW
    pltpu.sync_copy(in_hbm.at[pl.ds(base, _RPW), pl.ds(_ACQ, 1), :], buf_v)
    pltpu.sync_copy(buf_v, out_hbm.at[pl.ds(base, _RPW)])


def kernel(inputs):
    return _split(inputs).reshape(_B, _D)
